# baseline (speedup 1.0000x reference)
import functools

import jax
import jax.numpy as jnp
from jax import lax
from jax.experimental import pallas as pl
from jax.experimental.pallas import tpu as pltpu
from jax.experimental.pallas import tpu_sc as plsc

D = 256
HALF = 128
N_NODES = 10000
N_EDGES = 160000
N_GRAPHS = 64

NS = 16
E_PER_SUB = N_EDGES // NS
E_CHUNK = 80
N_CHUNKS = E_PER_SUB // E_CHUNK
NBUF_G = 2
NBUF_E = 2
NBUF_I = 4
ROWS_PER_SUB = 640
ACC_ROWS = ROWS_PER_SUB * NS
LANE = 16
ROW_CHUNK = 80


def _sc_aggregate(node_lo, node_hi, edge_feats, src_r, dst_r):
    mesh = plsc.VectorSubcoreMesh(core_axis_name="c", subcore_axis_name="s")

    @functools.partial(
        pl.kernel,
        out_type=(
            jax.ShapeDtypeStruct((N_NODES, HALF), jnp.float32),
            jax.ShapeDtypeStruct((N_NODES, HALF), jnp.float32),
        ),
        mesh=mesh,
        scratch_types=[
            pltpu.VMEM_SHARED((ACC_ROWS, HALF), jnp.float32),
            pltpu.VMEM((NBUF_I, E_CHUNK), jnp.int32),
            pltpu.VMEM((NBUF_I, E_CHUNK), jnp.int32),
            pltpu.VMEM((NBUF_E, E_CHUNK, HALF), jnp.float32),
            pltpu.VMEM((NBUF_G, E_CHUNK, HALF), jnp.float32),
            pltpu.SemaphoreType.DMA((NBUF_E,)),
            pltpu.SemaphoreType.DMA((NBUF_G,)),
            pltpu.SemaphoreType.DMA((NBUF_G,)),
            pltpu.SemaphoreType.DMA((NBUF_I,)),
            pltpu.SemaphoreType.DMA((NBUF_I,)),
        ],
    )
    def sc_kernel(nlo, nhi, ef, src_hbm, dst_hbm, out_lo, out_hi, acc, sring,
                  didx, ebuf, gbuf, sem_e, sem_g, sem_s, sem_d, sem_si):
        c = lax.axis_index("c")
        s = lax.axis_index("s")
        base_row = s * ROWS_PER_SUB
        ebase = s * E_PER_SUB
        coff = c * HALF

        zeros = jnp.zeros((LANE,), jnp.float32)

        @plsc.parallel_loop(0, ROW_CHUNK, 1, unroll=4)
        def zero_row(r):
            for kk in range(HALF // LANE):
                gbuf[0, r, pl.ds(kk * LANE, LANE)] = zeros
        for i in range(ROWS_PER_SUB // ROW_CHUNK):
            pltpu.async_copy(gbuf.at[0, pl.ds(0, ROW_CHUNK)],
                             acc.at[pl.ds(base_row + i * ROW_CHUNK,
                                          ROW_CHUNK)],
                             sem_g.at[0])
        for i in range(ROWS_PER_SUB // ROW_CHUNK):
            pltpu.make_async_copy(gbuf.at[0, pl.ds(0, ROW_CHUNK)],
                                  acc.at[pl.ds(base_row, ROW_CHUNK)],
                                  sem_g.at[0]).wait()
        plsc.subcore_barrier()

        def edge_copy(j, be):
            return pltpu.make_async_copy(
                ef.at[pl.ds(ebase + j * E_CHUNK, E_CHUNK),
                      pl.ds(coff, HALF)],
                ebuf.at[be], sem_e.at[be])

        def didx_copy(j, bi):
            return pltpu.make_async_copy(dst_hbm.at[s, j], didx.at[bi],
                                         sem_d.at[bi])

        def sidx_copy(j, bi):
            return pltpu.make_async_copy(src_hbm.at[s, j], sring.at[bi],
                                         sem_si.at[bi])

        def gather_copy_lo(bg, bi):
            return pltpu.make_async_copy(
                nlo.at[sring.at[bi]], gbuf.at[bg], sem_g.at[bg])

        def gather_copy_hi(bg, bi):
            return pltpu.make_async_copy(
                nhi.at[sring.at[bi]], gbuf.at[bg], sem_g.at[bg])

        def scatter_copy(bg, bi):
            return pltpu.make_async_copy(gbuf.at[bg], acc.at[didx.at[bi]],
                                         sem_s.at[bg])

        def start_gather(bg, bi):
            @pl.when(c == 0)
            def _():
                gather_copy_lo(bg, bi).start()

            @pl.when(c != 0)
            def _():
                gather_copy_hi(bg, bi).start()

        def wait_gather(bg, bi):
            @pl.when(c == 0)
            def _():
                gather_copy_lo(bg, bi).wait()

            @pl.when(c != 0)
            def _():
                gather_copy_hi(bg, bi).wait()

        def stage_idx(j, bi):
            didx_copy(j, bi).start()
            sidx_copy(j, bi).start()

        stage_idx(0, 0)
        stage_idx(1, 1)
        stage_idx(2, 2)
        edge_copy(0, 0).start()
        sidx_copy(0, 0).wait()
        start_gather(0, 0)

        def body(j, carry):
            be = lax.rem(j, NBUF_E)
            bg = lax.rem(j, NBUF_G)
            bi = lax.rem(j, NBUF_I)
            jn = j + 1
            j3 = j + 3

            @pl.when(j >= 1)
            def _():
                scatter_copy(lax.rem(jn, NBUF_G), lax.rem(j3, NBUF_I)).wait()

            @pl.when(j3 < N_CHUNKS)
            def _():
                stage_idx(j3, lax.rem(j3, NBUF_I))

            @pl.when(jn < N_CHUNKS)
            def _():
                bni = lax.rem(jn, NBUF_I)
                sidx_copy(jn, bni).wait()
                start_gather(lax.rem(jn, NBUF_G), bni)
                edge_copy(jn, lax.rem(jn, NBUF_E)).start()

            didx_copy(j, bi).wait()
            edge_copy(j, be).wait()
            wait_gather(bg, bi)

            @plsc.parallel_loop(0, E_CHUNK, 1, unroll=8)
            def relu_row(r):
                for kk in range(HALF // LANE):
                    sl = pl.ds(kk * LANE, LANE)
                    gbuf[bg, r, sl] = jnp.maximum(
                        gbuf[bg, r, sl] + ebuf[be, r, sl], 0.0)
            pltpu.async_copy(gbuf.at[bg], acc.at[didx.at[bi]],
                             sem_s.at[bg], add=True)
            return carry

        lax.fori_loop(0, N_CHUNKS, body, 0)
        scatter_copy((N_CHUNKS - 1) % NBUF_G, (N_CHUNKS - 1) % NBUF_I).wait()
        plsc.subcore_barrier()

        last_rows = N_NODES - (NS - 1) * ROWS_PER_SUB

        def copy_out(out_ref):
            @pl.when(s < NS - 1)
            def _():
                pltpu.sync_copy(acc.at[pl.ds(base_row, ROWS_PER_SUB)],
                                out_ref.at[pl.ds(base_row, ROWS_PER_SUB)])

            @pl.when(s == NS - 1)
            def _():
                pltpu.sync_copy(acc.at[pl.ds(base_row, last_rows)],
                                out_ref.at[pl.ds(base_row, last_rows)])

        @pl.when(c == 0)
        def _():
            copy_out(out_lo)

        @pl.when(c != 0)
        def _():
            copy_out(out_hi)

    return sc_kernel(node_lo, node_hi, edge_feats, src_r, dst_r)


ROW_BLK = 1000


def _tc_block(node_ref, alo_ref, ahi_ref, w1_ref, b1_ref, w2_ref, b2_ref,
              g_ref, bt_ref, ids_full_ref, ids_blk_ref, out_ref, inv_ref):
    gids = lax.broadcasted_iota(jnp.int32, (1, N_GRAPHS), 1)

    @pl.when(pl.program_id(0) == 0)
    def _():
        counts = jnp.sum((ids_full_ref[...] == gids).astype(jnp.float32),
                         axis=0, keepdims=True)
        inv_ref[0:1, 0:N_GRAPHS] = lax.rsqrt(jnp.maximum(counts, 1.0))

    x = node_ref[...]
    agg = jnp.concatenate([alo_ref[...], ahi_ref[...]], axis=1)
    h = x + agg
    h = jnp.dot(h, w1_ref[...], preferred_element_type=jnp.float32)
    h = h + b1_ref[...]
    h = jnp.where(h > 0, h, 0.2 * h)
    h = jnp.dot(h, w2_ref[...], preferred_element_type=jnp.float32)
    h = h + b2_ref[...]
    mean = jnp.mean(h, axis=-1, keepdims=True)
    var = jnp.mean((h - mean) ** 2, axis=-1, keepdims=True)
    h = (h - mean) * lax.rsqrt(var + 1e-5)
    h = h * g_ref[...] + bt_ref[...]
    inv = inv_ref[0:1, 0:N_GRAPHS]
    oh = (ids_blk_ref[...] == gids).astype(jnp.float32)
    scale = jnp.sum(oh * inv, axis=1, keepdims=True)
    h = h * scale
    h = jnp.where(h > 0, h, 0.2 * h)
    out_ref[...] = h + x


def _tc_post(node_feats, agg_lo, agg_hi, W1, b1, W2, b2, ln_gamma, ln_beta,
             node_graph_ids):
    ids2d = node_graph_ids.reshape(N_NODES, 1)
    grid = N_NODES // ROW_BLK
    return pl.pallas_call(
        _tc_block,
        grid=(grid,),
        in_specs=[
            pl.BlockSpec((ROW_BLK, D), lambda i: (i, 0)),
            pl.BlockSpec((ROW_BLK, HALF), lambda i: (i, 0)),
            pl.BlockSpec((ROW_BLK, HALF), lambda i: (i, 0)),
            pl.BlockSpec((D, 2 * D), lambda i: (0, 0)),
            pl.BlockSpec((1, 2 * D), lambda i: (0, 0)),
            pl.BlockSpec((2 * D, D), lambda i: (0, 0)),
            pl.BlockSpec((1, D), lambda i: (0, 0)),
            pl.BlockSpec((1, D), lambda i: (0, 0)),
            pl.BlockSpec((1, D), lambda i: (0, 0)),
            pl.BlockSpec((N_NODES, 1), lambda i: (0, 0)),
            pl.BlockSpec((ROW_BLK, 1), lambda i: (i, 0)),
        ],
        out_specs=pl.BlockSpec((ROW_BLK, D), lambda i: (i, 0)),
        out_shape=jax.ShapeDtypeStruct((N_NODES, D), jnp.float32),
        scratch_shapes=[pltpu.VMEM((8, 128), jnp.float32)],
    )(node_feats, agg_lo, agg_hi, W1, b1.reshape(1, -1), W2,
      b2.reshape(1, -1), ln_gamma.reshape(1, -1), ln_beta.reshape(1, -1),
      ids2d, ids2d)


def kernel(node_feats, edge_feats, W1, b1, W2, b2, ln_gamma, ln_beta,
           edge_index, node_graph_ids):
    node_lo = lax.slice(node_feats, (0, 0), (N_NODES, HALF))
    node_hi = lax.slice(node_feats, (0, HALF), (N_NODES, D))
    src_r = edge_index[0].reshape(NS, N_CHUNKS, E_CHUNK)
    dst_r = edge_index[1].reshape(NS, N_CHUNKS, E_CHUNK)
    agg_lo, agg_hi = _sc_aggregate(node_lo, node_hi, edge_feats, src_r,
                                   dst_r)
    return _tc_post(node_feats, agg_lo, agg_hi, W1, b1, W2, b2, ln_gamma,
                    ln_beta, node_graph_ids)


# speedup vs baseline: 1.0345x; 1.0345x over previous
import functools

import jax
import jax.numpy as jnp
from jax import lax
from jax.experimental import pallas as pl
from jax.experimental.pallas import tpu as pltpu
from jax.experimental.pallas import tpu_sc as plsc

D = 256
HALF = 128
N_NODES = 10000
N_EDGES = 160000
N_GRAPHS = 64

NS = 16
E_PER_SUB = N_EDGES // NS
E_CHUNK = 40
N_CHUNKS = E_PER_SUB // E_CHUNK
NBUF_G = 5
NBUF_E = 4
NBUF_I = 7
PF = 3
PFI = 5
ROWS_PER_SUB = 640
ACC_ROWS = ROWS_PER_SUB * NS
LANE = 16
ROW_CHUNK = 40


def _sc_aggregate(node_lo, node_hi, edge_feats, src_r, dst_r):
    mesh = plsc.VectorSubcoreMesh(core_axis_name="c", subcore_axis_name="s")

    @functools.partial(
        pl.kernel,
        out_type=(
            jax.ShapeDtypeStruct((N_NODES, HALF), jnp.float32),
            jax.ShapeDtypeStruct((N_NODES, HALF), jnp.float32),
        ),
        mesh=mesh,
        scratch_types=[
            pltpu.VMEM_SHARED((ACC_ROWS, HALF), jnp.float32),
            pltpu.VMEM((NBUF_I, E_CHUNK), jnp.int32),
            pltpu.VMEM((NBUF_I, E_CHUNK), jnp.int32),
            pltpu.VMEM((NBUF_E, E_CHUNK, HALF), jnp.float32),
            pltpu.VMEM((NBUF_G, E_CHUNK, HALF), jnp.float32),
            pltpu.SemaphoreType.DMA((NBUF_E,)),
            pltpu.SemaphoreType.DMA((NBUF_G,)),
            pltpu.SemaphoreType.DMA((NBUF_G,)),
            pltpu.SemaphoreType.DMA((NBUF_I,)),
            pltpu.SemaphoreType.DMA((NBUF_I,)),
        ],
    )
    def sc_kernel(nlo, nhi, ef, src_hbm, dst_hbm, out_lo, out_hi, acc, sring,
                  didx, ebuf, gbuf, sem_e, sem_g, sem_s, sem_d, sem_si):
        c = lax.axis_index("c")
        s = lax.axis_index("s")
        base_row = s * ROWS_PER_SUB
        ebase = s * E_PER_SUB
        coff = c * HALF

        zeros = jnp.zeros((LANE,), jnp.float32)

        @plsc.parallel_loop(0, ROW_CHUNK, 1, unroll=4)
        def zero_row(r):
            for kk in range(HALF // LANE):
                gbuf[0, r, pl.ds(kk * LANE, LANE)] = zeros
        for i in range(ROWS_PER_SUB // ROW_CHUNK):
            pltpu.async_copy(gbuf.at[0, pl.ds(0, ROW_CHUNK)],
                             acc.at[pl.ds(base_row + i * ROW_CHUNK,
                                          ROW_CHUNK)],
                             sem_g.at[0])
        for i in range(ROWS_PER_SUB // ROW_CHUNK):
            pltpu.make_async_copy(gbuf.at[0, pl.ds(0, ROW_CHUNK)],
                                  acc.at[pl.ds(base_row, ROW_CHUNK)],
                                  sem_g.at[0]).wait()
        plsc.subcore_barrier()

        def edge_copy(j, be):
            return pltpu.make_async_copy(
                ef.at[pl.ds(ebase + j * E_CHUNK, E_CHUNK),
                      pl.ds(coff, HALF)],
                ebuf.at[be], sem_e.at[be])

        def didx_copy(j, bi):
            return pltpu.make_async_copy(dst_hbm.at[s, j], didx.at[bi],
                                         sem_d.at[bi])

        def sidx_copy(j, bi):
            return pltpu.make_async_copy(src_hbm.at[s, j], sring.at[bi],
                                         sem_si.at[bi])

        def gather_copy_lo(bg, bi):
            return pltpu.make_async_copy(
                nlo.at[sring.at[bi]], gbuf.at[bg], sem_g.at[bg])

        def gather_copy_hi(bg, bi):
            return pltpu.make_async_copy(
                nhi.at[sring.at[bi]], gbuf.at[bg], sem_g.at[bg])

        def scatter_copy(bg, bi):
            return pltpu.make_async_copy(gbuf.at[bg], acc.at[didx.at[bi]],
                                         sem_s.at[bg])

        def start_gather(bg, bi):
            @pl.when(c == 0)
            def _():
                gather_copy_lo(bg, bi).start()

            @pl.when(c != 0)
            def _():
                gather_copy_hi(bg, bi).start()

        def wait_gather(bg, bi):
            @pl.when(c == 0)
            def _():
                gather_copy_lo(bg, bi).wait()

            @pl.when(c != 0)
            def _():
                gather_copy_hi(bg, bi).wait()

        def stage_idx(j, bi):
            didx_copy(j, bi).start()
            sidx_copy(j, bi).start()

        for k in range(PFI):
            stage_idx(k, k)
        for k in range(PF):
            sidx_copy(k, k).wait()
            start_gather(k % NBUF_G, k % NBUF_I)
            edge_copy(k, k % NBUF_E).start()

        def body(j, carry):
            be = lax.rem(j, NBUF_E)
            bg = lax.rem(j, NBUF_G)
            bi = lax.rem(j, NBUF_I)

            @pl.when(j >= 2)
            def _():
                scatter_copy(lax.rem(j + NBUF_G - 2, NBUF_G),
                             lax.rem(j + NBUF_I - 2, NBUF_I)).wait()

            @pl.when(j + PFI < N_CHUNKS)
            def _():
                stage_idx(j + PFI, lax.rem(j + PFI, NBUF_I))

            @pl.when(j + PF < N_CHUNKS)
            def _():
                bni = lax.rem(j + PF, NBUF_I)
                sidx_copy(j + PF, bni).wait()
                start_gather(lax.rem(j + PF, NBUF_G), bni)
                edge_copy(j + PF, lax.rem(j + PF, NBUF_E)).start()

            didx_copy(j, bi).wait()
            edge_copy(j, be).wait()
            wait_gather(bg, bi)

            @plsc.parallel_loop(0, E_CHUNK, 1, unroll=8)
            def relu_row(r):
                for kk in range(HALF // LANE):
                    sl = pl.ds(kk * LANE, LANE)
                    gbuf[bg, r, sl] = jnp.maximum(
                        gbuf[bg, r, sl] + ebuf[be, r, sl], 0.0)
            pltpu.async_copy(gbuf.at[bg], acc.at[didx.at[bi]],
                             sem_s.at[bg], add=True)
            return carry

        lax.fori_loop(0, N_CHUNKS, body, 0)
        for k in (N_CHUNKS - 2, N_CHUNKS - 1):
            scatter_copy(k % NBUF_G, k % NBUF_I).wait()
        plsc.subcore_barrier()

        last_rows = N_NODES - (NS - 1) * ROWS_PER_SUB

        def copy_out(out_ref):
            @pl.when(s < NS - 1)
            def _():
                pltpu.sync_copy(acc.at[pl.ds(base_row, ROWS_PER_SUB)],
                                out_ref.at[pl.ds(base_row, ROWS_PER_SUB)])

            @pl.when(s == NS - 1)
            def _():
                pltpu.sync_copy(acc.at[pl.ds(base_row, last_rows)],
                                out_ref.at[pl.ds(base_row, last_rows)])

        @pl.when(c == 0)
        def _():
            copy_out(out_lo)

        @pl.when(c != 0)
        def _():
            copy_out(out_hi)

    return sc_kernel(node_lo, node_hi, edge_feats, src_r, dst_r)


ROW_BLK = 1000


def _tc_block(node_ref, alo_ref, ahi_ref, w1_ref, b1_ref, w2_ref, b2_ref,
              g_ref, bt_ref, ids_full_ref, ids_blk_ref, out_ref, inv_ref):
    gids = lax.broadcasted_iota(jnp.int32, (1, N_GRAPHS), 1)

    @pl.when(pl.program_id(0) == 0)
    def _():
        counts = jnp.sum((ids_full_ref[...] == gids).astype(jnp.float32),
                         axis=0, keepdims=True)
        inv_ref[0:1, 0:N_GRAPHS] = lax.rsqrt(jnp.maximum(counts, 1.0))

    x = node_ref[...]
    agg = jnp.concatenate([alo_ref[...], ahi_ref[...]], axis=1)
    h = x + agg
    h = jnp.dot(h, w1_ref[...], preferred_element_type=jnp.float32)
    h = h + b1_ref[...]
    h = jnp.where(h > 0, h, 0.2 * h)
    h = jnp.dot(h, w2_ref[...], preferred_element_type=jnp.float32)
    h = h + b2_ref[...]
    mean = jnp.mean(h, axis=-1, keepdims=True)
    var = jnp.mean((h - mean) ** 2, axis=-1, keepdims=True)
    h = (h - mean) * lax.rsqrt(var + 1e-5)
    h = h * g_ref[...] + bt_ref[...]
    inv = inv_ref[0:1, 0:N_GRAPHS]
    oh = (ids_blk_ref[...] == gids).astype(jnp.float32)
    scale = jnp.sum(oh * inv, axis=1, keepdims=True)
    h = h * scale
    h = jnp.where(h > 0, h, 0.2 * h)
    out_ref[...] = h + x


def _tc_post(node_feats, agg_lo, agg_hi, W1, b1, W2, b2, ln_gamma, ln_beta,
             node_graph_ids):
    ids2d = node_graph_ids.reshape(N_NODES, 1)
    grid = N_NODES // ROW_BLK
    return pl.pallas_call(
        _tc_block,
        grid=(grid,),
        in_specs=[
            pl.BlockSpec((ROW_BLK, D), lambda i: (i, 0)),
            pl.BlockSpec((ROW_BLK, HALF), lambda i: (i, 0)),
            pl.BlockSpec((ROW_BLK, HALF), lambda i: (i, 0)),
            pl.BlockSpec((D, 2 * D), lambda i: (0, 0)),
            pl.BlockSpec((1, 2 * D), lambda i: (0, 0)),
            pl.BlockSpec((2 * D, D), lambda i: (0, 0)),
            pl.BlockSpec((1, D), lambda i: (0, 0)),
            pl.BlockSpec((1, D), lambda i: (0, 0)),
            pl.BlockSpec((1, D), lambda i: (0, 0)),
            pl.BlockSpec((N_NODES, 1), lambda i: (0, 0)),
            pl.BlockSpec((ROW_BLK, 1), lambda i: (i, 0)),
        ],
        out_specs=pl.BlockSpec((ROW_BLK, D), lambda i: (i, 0)),
        out_shape=jax.ShapeDtypeStruct((N_NODES, D), jnp.float32),
        scratch_shapes=[pltpu.VMEM((8, 128), jnp.float32)],
    )(node_feats, agg_lo, agg_hi, W1, b1.reshape(1, -1), W2,
      b2.reshape(1, -1), ln_gamma.reshape(1, -1), ln_beta.reshape(1, -1),
      ids2d, ids2d)


def kernel(node_feats, edge_feats, W1, b1, W2, b2, ln_gamma, ln_beta,
           edge_index, node_graph_ids):
    node_lo = lax.slice(node_feats, (0, 0), (N_NODES, HALF))
    node_hi = lax.slice(node_feats, (0, HALF), (N_NODES, D))
    src_r = edge_index[0].reshape(NS, N_CHUNKS, E_CHUNK)
    dst_r = edge_index[1].reshape(NS, N_CHUNKS, E_CHUNK)
    agg_lo, agg_hi = _sc_aggregate(node_lo, node_hi, edge_feats, src_r,
                                   dst_r)
    return _tc_post(node_feats, agg_lo, agg_hi, W1, b1, W2, b2, ln_gamma,
                    ln_beta, node_graph_ids)


# speedup vs baseline: 1.1136x; 1.0764x over previous
import functools

import jax
import jax.numpy as jnp
from jax import lax
from jax.experimental import pallas as pl
from jax.experimental.pallas import tpu as pltpu
from jax.experimental.pallas import tpu_sc as plsc

D = 256
HALF = 128
N_NODES = 10000
N_EDGES = 160000
N_GRAPHS = 64

NS = 16
E_PER_SUB = N_EDGES // NS
E_CHUNK = 40
N_CHUNKS = E_PER_SUB // E_CHUNK
NBUF_G = 4
NBUF_E = 3
ROWS_PER_SUB = 640
ACC_ROWS = ROWS_PER_SUB * NS
LANE = 16
ROW_CHUNK = 40


def _sc_aggregate(node_lo, node_hi, edge_feats, src_r, dst_r):
    mesh = plsc.VectorSubcoreMesh(core_axis_name="c", subcore_axis_name="s")

    @functools.partial(
        pl.kernel,
        out_type=(
            jax.ShapeDtypeStruct((N_NODES, HALF), jnp.float32),
            jax.ShapeDtypeStruct((N_NODES, HALF), jnp.float32),
        ),
        mesh=mesh,
        scratch_types=[
            pltpu.VMEM_SHARED((ACC_ROWS, HALF), jnp.float32),
            pltpu.VMEM((E_PER_SUB,), jnp.int32),
            pltpu.VMEM((NBUF_G, E_CHUNK), jnp.int32),
            pltpu.VMEM((NBUF_E, E_CHUNK, HALF), jnp.float32),
            pltpu.VMEM((NBUF_G, E_CHUNK, HALF), jnp.float32),
            pltpu.SemaphoreType.DMA((NBUF_E,)),
            pltpu.SemaphoreType.DMA((NBUF_G,)),
            pltpu.SemaphoreType.DMA((NBUF_G,)),
            pltpu.SemaphoreType.DMA((NBUF_G,)),
            pltpu.SemaphoreType.DMA,
        ],
    )
    def sc_kernel(nlo, nhi, ef, src_hbm, dst_hbm, out_lo, out_hi, acc, sidx,
                  didx, ebuf, gbuf, sem_e, sem_g, sem_s, sem_d, sem_si):
        c = lax.axis_index("c")
        s = lax.axis_index("s")
        base_row = s * ROWS_PER_SUB
        ebase = s * E_PER_SUB
        coff = c * HALF

        def edge_copy(j, be):
            return pltpu.make_async_copy(
                ef.at[pl.ds(ebase + j * E_CHUNK, E_CHUNK),
                      pl.ds(coff, HALF)],
                ebuf.at[be], sem_e.at[be])

        def didx_copy(j, bg):
            return pltpu.make_async_copy(dst_hbm.at[s, j], didx.at[bg],
                                         sem_d.at[bg])

        def gather_copy_lo(j, bg):
            return pltpu.make_async_copy(
                nlo.at[sidx.at[pl.ds(j * E_CHUNK, E_CHUNK)]], gbuf.at[bg],
                sem_g.at[bg])

        def gather_copy_hi(j, bg):
            return pltpu.make_async_copy(
                nhi.at[sidx.at[pl.ds(j * E_CHUNK, E_CHUNK)]], gbuf.at[bg],
                sem_g.at[bg])

        def scatter_copy(bg):
            return pltpu.make_async_copy(gbuf.at[bg], acc.at[didx.at[bg]],
                                         sem_s.at[bg])

        def start_gather(j, bg):
            @pl.when(c == 0)
            def _():
                gather_copy_lo(j, bg).start()

            @pl.when(c != 0)
            def _():
                gather_copy_hi(j, bg).start()

        def stage_in(j, be, bg):
            didx_copy(j, bg).start()
            edge_copy(j, be).start()
            start_gather(j, bg)

        pltpu.make_async_copy(src_hbm.at[s], sidx, sem_si).start()
        didx_copy(0, 0).start()
        edge_copy(0, 0).start()
        didx_copy(1, 1).start()
        edge_copy(1, 1).start()

        zeros = jnp.zeros((LANE,), jnp.float32)

        @plsc.parallel_loop(0, ROW_CHUNK, 1, unroll=4)
        def zero_row(r):
            for kk in range(HALF // LANE):
                gbuf[0, r, pl.ds(kk * LANE, LANE)] = zeros
        for i in range(ROWS_PER_SUB // ROW_CHUNK):
            pltpu.async_copy(gbuf.at[0, pl.ds(0, ROW_CHUNK)],
                             acc.at[pl.ds(base_row + i * ROW_CHUNK,
                                          ROW_CHUNK)],
                             sem_g.at[0])

        pltpu.make_async_copy(src_hbm.at[s], sidx, sem_si).wait()
        start_gather(1, 1)
        for i in range(ROWS_PER_SUB // ROW_CHUNK):
            pltpu.make_async_copy(gbuf.at[0, pl.ds(0, ROW_CHUNK)],
                                  acc.at[pl.ds(base_row, ROW_CHUNK)],
                                  sem_g.at[0]).wait()
        start_gather(0, 0)
        plsc.subcore_barrier()

        def body(j, carry):
            be = lax.rem(j, NBUF_E)
            bg = lax.rem(j, NBUF_G)
            jn = j + 2

            @pl.when(jn < N_CHUNKS)
            def _():
                bng = lax.rem(jn, NBUF_G)

                @pl.when(j >= 2)
                def _():
                    scatter_copy(bng).wait()

                stage_in(jn, lax.rem(jn, NBUF_E), bng)

            didx_copy(j, bg).wait()
            edge_copy(j, be).wait()

            @pl.when(c == 0)
            def _():
                gather_copy_lo(j, bg).wait()

            @pl.when(c != 0)
            def _():
                gather_copy_hi(j, bg).wait()

            @plsc.parallel_loop(0, E_CHUNK, 1, unroll=8)
            def relu_row(r):
                for kk in range(HALF // LANE):
                    sl = pl.ds(kk * LANE, LANE)
                    gbuf[bg, r, sl] = jnp.maximum(
                        gbuf[bg, r, sl] + ebuf[be, r, sl], 0.0)
            pltpu.async_copy(gbuf.at[bg], acc.at[didx.at[bg]],
                             sem_s.at[bg], add=True)
            return carry

        lax.fori_loop(0, N_CHUNKS, body, 0)
        for k in range(NBUF_G):
            scatter_copy(k).wait()
        plsc.subcore_barrier()

        last_rows = N_NODES - (NS - 1) * ROWS_PER_SUB

        def copy_out(out_ref):
            @pl.when(s < NS - 1)
            def _():
                pltpu.sync_copy(acc.at[pl.ds(base_row, ROWS_PER_SUB)],
                                out_ref.at[pl.ds(base_row, ROWS_PER_SUB)])

            @pl.when(s == NS - 1)
            def _():
                pltpu.sync_copy(acc.at[pl.ds(base_row, last_rows)],
                                out_ref.at[pl.ds(base_row, last_rows)])

        @pl.when(c == 0)
        def _():
            copy_out(out_lo)

        @pl.when(c != 0)
        def _():
            copy_out(out_hi)

    return sc_kernel(node_lo, node_hi, edge_feats, src_r, dst_r)


ROW_BLK = 1000


def _tc_block(node_ref, alo_ref, ahi_ref, w1_ref, b1_ref, w2_ref, b2_ref,
              g_ref, bt_ref, ids_full_ref, ids_blk_ref, out_ref, inv_ref):
    gids = lax.broadcasted_iota(jnp.int32, (1, N_GRAPHS), 1)

    @pl.when(pl.program_id(0) == 0)
    def _():
        counts = jnp.sum((ids_full_ref[...] == gids).astype(jnp.float32),
                         axis=0, keepdims=True)
        inv_ref[0:1, 0:N_GRAPHS] = lax.rsqrt(jnp.maximum(counts, 1.0))

    x = node_ref[...]
    agg = jnp.concatenate([alo_ref[...], ahi_ref[...]], axis=1)
    h = x + agg
    h = jnp.dot(h, w1_ref[...], preferred_element_type=jnp.float32)
    h = h + b1_ref[...]
    h = jnp.where(h > 0, h, 0.2 * h)
    h = jnp.dot(h, w2_ref[...], preferred_element_type=jnp.float32)
    h = h + b2_ref[...]
    mean = jnp.mean(h, axis=-1, keepdims=True)
    var = jnp.mean((h - mean) ** 2, axis=-1, keepdims=True)
    h = (h - mean) * lax.rsqrt(var + 1e-5)
    h = h * g_ref[...] + bt_ref[...]
    inv = inv_ref[0:1, 0:N_GRAPHS]
    oh = (ids_blk_ref[...] == gids).astype(jnp.float32)
    scale = jnp.sum(oh * inv, axis=1, keepdims=True)
    h = h * scale
    h = jnp.where(h > 0, h, 0.2 * h)
    out_ref[...] = h + x


def _tc_post(node_feats, agg_lo, agg_hi, W1, b1, W2, b2, ln_gamma, ln_beta,
             node_graph_ids):
    ids2d = node_graph_ids.reshape(N_NODES, 1)
    grid = N_NODES // ROW_BLK
    return pl.pallas_call(
        _tc_block,
        grid=(grid,),
        in_specs=[
            pl.BlockSpec((ROW_BLK, D), lambda i: (i, 0)),
            pl.BlockSpec((ROW_BLK, HALF), lambda i: (i, 0)),
            pl.BlockSpec((ROW_BLK, HALF), lambda i: (i, 0)),
            pl.BlockSpec((D, 2 * D), lambda i: (0, 0)),
            pl.BlockSpec((1, 2 * D), lambda i: (0, 0)),
            pl.BlockSpec((2 * D, D), lambda i: (0, 0)),
            pl.BlockSpec((1, D), lambda i: (0, 0)),
            pl.BlockSpec((1, D), lambda i: (0, 0)),
            pl.BlockSpec((1, D), lambda i: (0, 0)),
            pl.BlockSpec((N_NODES, 1), lambda i: (0, 0)),
            pl.BlockSpec((ROW_BLK, 1), lambda i: (i, 0)),
        ],
        out_specs=pl.BlockSpec((ROW_BLK, D), lambda i: (i, 0)),
        out_shape=jax.ShapeDtypeStruct((N_NODES, D), jnp.float32),
        scratch_shapes=[pltpu.VMEM((8, 128), jnp.float32)],
    )(node_feats, agg_lo, agg_hi, W1, b1.reshape(1, -1), W2,
      b2.reshape(1, -1), ln_gamma.reshape(1, -1), ln_beta.reshape(1, -1),
      ids2d, ids2d)


def kernel(node_feats, edge_feats, W1, b1, W2, b2, ln_gamma, ln_beta,
           edge_index, node_graph_ids):
    node_lo = lax.slice(node_feats, (0, 0), (N_NODES, HALF))
    node_hi = lax.slice(node_feats, (0, HALF), (N_NODES, D))
    src_r = edge_index[0].reshape(NS, E_PER_SUB)
    dst_r = edge_index[1].reshape(NS, N_CHUNKS, E_CHUNK)
    agg_lo, agg_hi = _sc_aggregate(node_lo, node_hi, edge_feats, src_r,
                                   dst_r)
    return _tc_post(node_feats, agg_lo, agg_hi, W1, b1, W2, b2, ln_gamma,
                    ln_beta, node_graph_ids)
